# Initial kernel scaffold; baseline (speedup 1.0000x reference)
import jax
import jax.numpy as jnp
from jax.experimental import pallas as pl


def kernel(x, edge_index, edge_type, W0, b0, W1, b1, W2, b2, W3, b3):
    raise NotImplementedError("write your pallas kernel here")


# speedup vs baseline: 7.6352x; 7.6352x over previous
import functools

import jax
import jax.numpy as jnp
from jax import lax
from jax.experimental import pallas as pl
from jax.experimental.pallas import tpu as pltpu
from jax.experimental.pallas import tpu_sc as plsc

N = 10000
E = 320000
D = 128
NT = 4
NPAD = 10240
NW = 32
EPW = E // NW
CH = 80
NCH = EPW // CH
RB = 400
NRB = N // RB

_mesh = plsc.VectorSubcoreMesh(core_axis_name="c", subcore_axis_name="s",
                               num_cores=2, num_subcores=16)
_sc_params = pltpu.CompilerParams(needs_layout_passes=False)


def _y_body(x_ref, w_ref, y_ref):
    y_ref[...] = jnp.dot(x_ref[...], w_ref[0], preferred_element_type=jnp.float32)


def _y_call(x, Wb):
    return pl.pallas_call(
        _y_body,
        grid=(NT, NRB),
        in_specs=[
            pl.BlockSpec((RB, D), lambda v, i: (i, 0)),
            pl.BlockSpec((1, D, D), lambda v, i: (v, 0, 0)),
        ],
        out_specs=pl.BlockSpec((RB, D), lambda v, i: (v * NRB + i, 0)),
        out_shape=jax.ShapeDtypeStruct((NT * N, D), jnp.float32),
    )(x, Wb)


@functools.partial(
    pl.kernel,
    out_type=jax.ShapeDtypeStruct((2, NT * NPAD), jnp.float32),
    mesh=_mesh,
    compiler_params=_sc_params,
    scratch_types=[
        pltpu.VMEM_SHARED((NT * NPAD,), jnp.float32),
        pltpu.VMEM((NT * NPAD // 16,), jnp.float32),
        pltpu.VMEM((CH,), jnp.int32),
        pltpu.VMEM((CH,), jnp.int32),
        pltpu.VMEM((CH,), jnp.int32),
        pltpu.VMEM((CH,), jnp.float32),
    ],
)
def _deg_kernel(dst_hbm, et_hbm, out_hbm, hist_s, zbuf, dbuf, tbuf, cidx, ones):
    c = lax.axis_index("c")
    s = lax.axis_index("s")
    wid = c * 16 + s
    seg = NT * NPAD // 16

    for i in range(seg // 16):
        zbuf[pl.ds(i * 16, 16)] = jnp.zeros((16,), jnp.float32)
    pltpu.sync_copy(zbuf, hist_s.at[pl.ds(s * seg, seg)])
    for i in range(CH // 16):
        ones[pl.ds(i * 16, 16)] = jnp.ones((16,), jnp.float32)
    plsc.subcore_barrier()

    def chunk(k, carry):
        base = wid * EPW + k * CH
        pltpu.sync_copy(dst_hbm.at[pl.ds(base, CH)], dbuf)
        pltpu.sync_copy(et_hbm.at[pl.ds(base, CH)], tbuf)
        for g in range(CH // 16):
            sl = pl.ds(g * 16, 16)
            cidx[sl] = tbuf[sl] * NPAD + dbuf[sl]
        pltpu.sync_copy(ones, hist_s.at[cidx], add=True)
        return carry

    lax.fori_loop(0, NCH, chunk, 0)
    plsc.subcore_barrier()
    pltpu.sync_copy(hist_s.at[pl.ds(s * seg, seg)],
                    out_hbm.at[c, pl.ds(s * seg, seg)])


@functools.partial(
    pl.kernel,
    out_type=jax.ShapeDtypeStruct((2, NPAD, D), jnp.float32),
    mesh=_mesh,
    compiler_params=_sc_params,
    scratch_types=[
        pltpu.VMEM_SHARED((NPAD, D), jnp.float32),
        pltpu.VMEM((CH, D), jnp.float32),
        pltpu.VMEM((CH,), jnp.int32),
        pltpu.VMEM((CH,), jnp.int32),
        pltpu.VMEM((CH,), jnp.int32),
        pltpu.VMEM((CH,), jnp.int32),
        pltpu.VMEM((CH,), jnp.int32),
        pltpu.VMEM((CH,), jnp.int32),
        pltpu.VMEM((CH,), jnp.float32),
    ],
)
def _scat_kernel(src_hbm, dst_hbm, et_hbm, c_hbm, y_hbm, out_hbm,
                 acc_s, rows, sbuf, dbuf, tbuf, yidx, didx, kidx, cch):
    c = lax.axis_index("c")
    s = lax.axis_index("s")
    wid = c * 16 + s
    for j in range(CH):
        for k in range(D // 16):
            rows[j, pl.ds(k * 16, 16)] = jnp.zeros((16,), jnp.float32)
    seg = NPAD // 16
    for z in range(seg // CH):
        pltpu.sync_copy(rows, acc_s.at[pl.ds(s * seg + z * CH, CH)])
    plsc.subcore_barrier()

    def chunk(k, carry):
        base = wid * EPW + k * CH
        pltpu.sync_copy(src_hbm.at[pl.ds(base, CH)], sbuf)
        pltpu.sync_copy(dst_hbm.at[pl.ds(base, CH)], dbuf)
        pltpu.sync_copy(et_hbm.at[pl.ds(base, CH)], tbuf)
        for g in range(CH // 16):
            sl = pl.ds(g * 16, 16)
            sv = sbuf[sl]
            dv = dbuf[sl]
            tv = tbuf[sl]
            yidx[sl] = tv * N + sv
            didx[sl] = dv
            kidx[sl] = tv * NPAD + dv
        pltpu.sync_copy(c_hbm.at[kidx], cch)
        pltpu.sync_copy(y_hbm.at[yidx], rows)
        for g in range(CH // 16):
            grp = cch[pl.ds(g * 16, 16)]
            for l in range(16):
                j = g * 16 + l
                cj = grp.at[jnp.full((16,), l, jnp.int32)].get(mode="promise_in_bounds")
                for k in range(D // 16):
                    sl = pl.ds(k * 16, 16)
                    rows[j, sl] = rows[j, sl] * cj
        pltpu.sync_copy(rows, acc_s.at[didx], add=True)
        return carry

    lax.fori_loop(0, NCH, chunk, 0)
    plsc.subcore_barrier()
    for z in range(seg // CH):
        st = s * seg + z * CH
        pltpu.sync_copy(acc_s.at[pl.ds(st, CH)], out_hbm.at[c, pl.ds(st, CH)])


def _comb_body(x_ref, wa_ref, ba_ref, a0_ref, a1_ref, o_ref):
    o_ref[...] = (jnp.dot(x_ref[...], wa_ref[...], preferred_element_type=jnp.float32)
                  + ba_ref[...] + a0_ref[...] + a1_ref[...])


def _comb_call(x, Wa, ba, a0, a1):
    blk = pl.BlockSpec((RB, D), lambda i: (i, 0))
    return pl.pallas_call(
        _comb_body,
        grid=(NRB,),
        in_specs=[
            blk,
            pl.BlockSpec((D, D), lambda i: (0, 0)),
            pl.BlockSpec((1, D), lambda i: (0, 0)),
            blk,
            blk,
        ],
        out_specs=blk,
        out_shape=jax.ShapeDtypeStruct((N, D), jnp.float32),
    )(x, Wa, ba, a0, a1)


def kernel(x, edge_index, edge_type, W0, b0, W1, b1, W2, b2, W3, b3):
    src = edge_index[0]
    dst = edge_index[1]
    Wb = jnp.stack([W0[D:], W1[D:], W2[D:], W3[D:]])
    Wa = (W0[:D] + W1[:D] + W2[:D] + W3[:D]) * 0.25
    ba = ((b0 + b1 + b2 + b3) * 0.25).reshape(1, D)

    Y = _y_call(x, Wb)
    degp = _deg_kernel(dst, edge_type)
    cvec = 0.25 / jnp.clip(degp[0] + degp[1], 1.0, None)
    accp = _scat_kernel(src, dst, edge_type, cvec, Y)
    return _comb_call(x, Wa, ba, accp[0, :N], accp[1, :N])


# speedup vs baseline: 17.6706x; 2.3144x over previous
import functools

import jax
import jax.numpy as jnp
from jax import lax
from jax.experimental import pallas as pl
from jax.experimental.pallas import tpu as pltpu
from jax.experimental.pallas import tpu_sc as plsc

N = 10000
E = 320000
D = 128
NT = 4
NPAD = 10240
NW = 32
EPW = E // NW
CH = 80
NCH = EPW // CH
RB = 400
NRB = N // RB

_mesh = plsc.VectorSubcoreMesh(core_axis_name="c", subcore_axis_name="s",
                               num_cores=2, num_subcores=16)
_sc_params = pltpu.CompilerParams(needs_layout_passes=False)


def _y_body(x_ref, w_ref, y_ref):
    y_ref[...] = jnp.dot(x_ref[...], w_ref[0], preferred_element_type=jnp.float32)


def _y_call(x, Wb):
    return pl.pallas_call(
        _y_body,
        grid=(NT, NRB),
        in_specs=[
            pl.BlockSpec((RB, D), lambda v, i: (i, 0)),
            pl.BlockSpec((1, D, D), lambda v, i: (v, 0, 0)),
        ],
        out_specs=pl.BlockSpec((RB, D), lambda v, i: (v * NRB + i, 0)),
        out_shape=jax.ShapeDtypeStruct((NT * N, D), jnp.float32),
    )(x, Wb)


@functools.partial(
    pl.kernel,
    out_type=jax.ShapeDtypeStruct((2, NT * NPAD), jnp.float32),
    mesh=_mesh,
    compiler_params=_sc_params,
    scratch_types=[
        pltpu.VMEM_SHARED((NT * NPAD,), jnp.float32),
        pltpu.VMEM((NT * NPAD // 16,), jnp.float32),
        pltpu.VMEM((3, 128), jnp.int32),
        pltpu.VMEM((3, 128), jnp.int32),
        pltpu.VMEM((CH,), jnp.int32),
        pltpu.VMEM((CH,), jnp.int32),
        pltpu.VMEM((CH,), jnp.float32),
        pltpu.SemaphoreType.DMA,
        pltpu.SemaphoreType.DMA,
        pltpu.SemaphoreType.DMA,
        pltpu.SemaphoreType.DMA,
    ],
)
def _deg_kernel(edges_hbm, out_hbm, hist_s, zbuf, ebuf0, ebuf1, cidx0, cidx1,
                ones, esem0, esem1, hsem0, hsem1):
    c = lax.axis_index("c")
    s = lax.axis_index("s")
    wid = c * 16 + s
    seg = NT * NPAD // 16
    esem = (esem0, esem1)
    hsem = (hsem0, hsem1)
    cidx = (cidx0, cidx1)
    ebuf = (ebuf0, ebuf1)

    for i in range(seg // 16):
        zbuf[pl.ds(i * 16, 16)] = jnp.zeros((16,), jnp.float32)
    pltpu.sync_copy(zbuf, hist_s.at[pl.ds(s * seg, seg)])
    for i in range(CH // 16):
        ones[pl.ds(i * 16, 16)] = jnp.ones((16,), jnp.float32)
    plsc.subcore_barrier()

    def issue_edges(k, b):
        pltpu.async_copy(edges_hbm.at[wid, k], ebuf[b], esem[b])

    def wait_edges(b):
        pltpu.make_async_copy(edges_hbm.at[wid, 0], ebuf[b], esem[b]).wait()

    def drain_hist(b):
        pltpu.make_async_copy(ones, hist_s.at[cidx[b]], hsem[b]).wait()

    def emit(k, b, guard):
        wait_edges(b)
        if guard is None:
            drain_hist(b)
        else:
            @pl.when(guard)
            def _():
                drain_hist(b)
        for g in range(CH // 16):
            sl = pl.ds(g * 16, 16)
            cidx[b][sl] = ebuf[b][2, sl] * NPAD + ebuf[b][1, sl]
        pltpu.async_copy(ones, hist_s.at[cidx[b]], hsem[b], add=True)
        if k is not None:
            @pl.when(k + 2 < NCH)
            def _():
                issue_edges(k + 2, b)

    issue_edges(0, 0)
    issue_edges(1, 1)

    def pair(i, carry):
        k0 = 2 * i
        emit(k0, 0, i > 0)
        emit(k0 + 1, 1, i > 0)
        return carry

    lax.fori_loop(0, (NCH - 1) // 2, pair, 0)
    emit(None, 0, None)
    pltpu.make_async_copy(ones, hist_s.at[cidx[0]], hsem[0]).wait()
    pltpu.make_async_copy(ones, hist_s.at[cidx[1]], hsem[1]).wait()
    plsc.subcore_barrier()
    pltpu.sync_copy(hist_s.at[pl.ds(s * seg, seg)],
                    out_hbm.at[c, pl.ds(s * seg, seg)])


@functools.partial(
    pl.kernel,
    out_type=jax.ShapeDtypeStruct((2, NPAD, D), jnp.float32),
    mesh=_mesh,
    compiler_params=_sc_params,
    scratch_types=[
        pltpu.VMEM_SHARED((NPAD, D), jnp.float32),
        pltpu.VMEM_SHARED((NT * NPAD,), jnp.float32),
        pltpu.VMEM((2, CH, D), jnp.float32),
        pltpu.VMEM((3, 128), jnp.int32),
        pltpu.VMEM((3, 128), jnp.int32),
        pltpu.VMEM((CH,), jnp.int32),
        pltpu.VMEM((CH,), jnp.int32),
        pltpu.VMEM((CH,), jnp.int32),
        pltpu.VMEM((CH,), jnp.int32),
        pltpu.VMEM((CH,), jnp.int32),
        pltpu.VMEM((CH,), jnp.int32),
        pltpu.VMEM((CH,), jnp.float32),
        pltpu.VMEM((CH,), jnp.float32),
        pltpu.VMEM((NT * NPAD // 16,), jnp.float32),
        pltpu.VMEM((NT * NPAD // 16,), jnp.float32),
        pltpu.SemaphoreType.DMA,
        pltpu.SemaphoreType.DMA,
        pltpu.SemaphoreType.DMA,
        pltpu.SemaphoreType.DMA,
        pltpu.SemaphoreType.DMA,
        pltpu.SemaphoreType.DMA,
        pltpu.SemaphoreType.DMA,
        pltpu.SemaphoreType.DMA,
    ],
)
def _scat_kernel(edges_hbm, deg_hbm, y_hbm, out_hbm,
                 acc_s, cs, rows, ebuf0, ebuf1, yidx0, yidx1, didx0, didx1,
                 kidx0, kidx1, cch0, cch1, dp0, dp1,
                 esem0, esem1, ysem0, ysem1, csem0, csem1, ssem0, ssem1):
    c = lax.axis_index("c")
    s = lax.axis_index("s")
    wid = c * 16 + s
    esem = (esem0, esem1)
    ysem = (ysem0, ysem1)
    csem = (csem0, csem1)
    ssem = (ssem0, ssem1)
    ebuf = (ebuf0, ebuf1)
    yidx = (yidx0, yidx1)
    didx = (didx0, didx1)
    kidx = (kidx0, kidx1)
    cch = (cch0, cch1)

    cseg = NT * NPAD // 16
    pltpu.sync_copy(deg_hbm.at[0, pl.ds(s * cseg, cseg)], dp0)
    pltpu.sync_copy(deg_hbm.at[1, pl.ds(s * cseg, cseg)], dp1)
    for i in range(cseg // 16):
        sl = pl.ds(i * 16, 16)
        dp0[sl] = 0.25 / jnp.maximum(dp0[sl] + dp1[sl], 1.0)
    pltpu.sync_copy(dp0, cs.at[pl.ds(s * cseg, cseg)])

    for j in range(CH):
        for k in range(D // 16):
            rows[0, j, pl.ds(k * 16, 16)] = jnp.zeros((16,), jnp.float32)
    seg = NPAD // 16
    for z in range(seg // CH):
        pltpu.sync_copy(rows.at[0], acc_s.at[pl.ds(s * seg + z * CH, CH)])
    plsc.subcore_barrier()

    def issue_edges(k, b):
        pltpu.async_copy(edges_hbm.at[wid, k], ebuf[b], esem[b])

    def wait_edges(b):
        pltpu.make_async_copy(edges_hbm.at[wid, 0], ebuf[b], esem[b]).wait()

    def drain_scatter(b):
        pltpu.make_async_copy(rows.at[b], acc_s.at[didx[b]], ssem[b]).wait()

    def prep(k, b, drain_guard, pref_guard):
        wait_edges(b)
        if drain_guard is None:
            drain_scatter(b)
        else:
            @pl.when(drain_guard)
            def _():
                drain_scatter(b)
        for g in range(CH // 16):
            sl = pl.ds(g * 16, 16)
            sv = ebuf[b][0, sl]
            dv = ebuf[b][1, sl]
            tv = ebuf[b][2, sl]
            yidx[b][sl] = tv * N + sv
            didx[b][sl] = dv
            kidx[b][sl] = tv * NPAD + dv
        pltpu.async_copy(cs.at[kidx[b]], cch[b], csem[b])
        pltpu.async_copy(y_hbm.at[yidx[b]], rows.at[b], ysem[b])
        if pref_guard is not None:
            @pl.when(pref_guard)
            def _():
                issue_edges(k + 2, b)

    def process(b):
        pltpu.make_async_copy(cs.at[kidx[b]], cch[b], csem[b]).wait()
        pltpu.make_async_copy(y_hbm.at[yidx[b]], rows.at[b], ysem[b]).wait()
        for g in range(CH // 16):
            grp = cch[b][pl.ds(g * 16, 16)]
            for l in range(16):
                j = g * 16 + l
                cj = grp.at[jnp.full((16,), l, jnp.int32)].get(
                    mode="promise_in_bounds")
                for kk in range(D // 16):
                    sl = pl.ds(kk * 16, 16)
                    rows[b, j, sl] = rows[b, j, sl] * cj
        pltpu.async_copy(rows.at[b], acc_s.at[didx[b]], ssem[b], add=True)

    issue_edges(0, 0)
    issue_edges(1, 1)
    wait_edges(0)
    for g in range(CH // 16):
        sl = pl.ds(g * 16, 16)
        sv = ebuf[0][0, sl]
        dv = ebuf[0][1, sl]
        tv = ebuf[0][2, sl]
        yidx[0][sl] = tv * N + sv
        didx[0][sl] = dv
        kidx[0][sl] = tv * NPAD + dv
    pltpu.async_copy(cs.at[kidx[0]], cch[0], csem[0])
    pltpu.async_copy(y_hbm.at[yidx[0]], rows.at[0], ysem[0])
    issue_edges(2, 0)

    def pair(i, carry):
        k0 = 2 * i
        prep(k0 + 1, 1, i > 0, k0 + 3 < NCH)
        process(0)
        prep(k0 + 2, 0, None, k0 + 4 < NCH)
        process(1)
        return carry

    lax.fori_loop(0, (NCH - 1) // 2, pair, 0)
    process(0)
    drain_scatter(1)
    drain_scatter(0)
    plsc.subcore_barrier()
    for z in range(seg // CH):
        st = s * seg + z * CH
        pltpu.sync_copy(acc_s.at[pl.ds(st, CH)], out_hbm.at[c, pl.ds(st, CH)])


def _comb_body(x_ref, wa_ref, ba_ref, a0_ref, a1_ref, o_ref):
    o_ref[...] = (jnp.dot(x_ref[...], wa_ref[...], preferred_element_type=jnp.float32)
                  + ba_ref[...] + a0_ref[...] + a1_ref[...])


def _comb_call(x, Wa, ba, a0, a1):
    blk = pl.BlockSpec((RB, D), lambda i: (i, 0))
    return pl.pallas_call(
        _comb_body,
        grid=(NRB,),
        in_specs=[
            blk,
            pl.BlockSpec((D, D), lambda i: (0, 0)),
            pl.BlockSpec((1, D), lambda i: (0, 0)),
            blk,
            blk,
        ],
        out_specs=blk,
        out_shape=jax.ShapeDtypeStruct((N, D), jnp.float32),
    )(x, Wa, ba, a0, a1)


def kernel(x, edge_index, edge_type, W0, b0, W1, b1, W2, b2, W3, b3):
    src = edge_index[0]
    dst = edge_index[1]
    Wb = jnp.stack([W0[D:], W1[D:], W2[D:], W3[D:]])
    Wa = (W0[:D] + W1[:D] + W2[:D] + W3[:D]) * 0.25
    ba = ((b0 + b1 + b2 + b3) * 0.25).reshape(1, D)
    packed = (jnp.stack([src, dst, edge_type])
              .reshape(3, NW, NCH, CH).transpose(1, 2, 0, 3))
    packed = jnp.pad(packed, ((0, 0), (0, 0), (0, 0), (0, 128 - CH)))

    Y = _y_call(x, Wb)
    degp = _deg_kernel(packed)
    accp = _scat_kernel(packed, degp, Y)
    return _comb_call(x, Wa, ba, accp[0, :N], accp[1, :N])
